# Initial kernel scaffold; baseline (speedup 1.0000x reference)
import jax
import jax.numpy as jnp
from jax.experimental import pallas as pl


def kernel(X, P_logit, Q):
    raise NotImplementedError("write your pallas kernel here")


# speedup vs baseline: 32.1517x; 32.1517x over previous
import jax
import jax.numpy as jnp
from jax.experimental import pallas as pl
from jax.experimental.pallas import tpu as pltpu

KTAPS = 20
KB = 10
NG = 2
ROWPAD = 24
AA = 21
UU = 100
LL = 334
PP = LL - KTAPS + 1
FF = 6


def _prep_kernel(pl_ref, q_ref, r_ref):
    pv = pl_ref[...]
    q = q_ref[...]
    meanq = jnp.mean(q)
    eps = jnp.exp(-jnp.log(1.0 / meanq))
    m = jnp.max(pv, axis=1, keepdims=True)
    e = jnp.exp(pv - m)
    p = e / jnp.sum(e, axis=1, keepdims=True)
    ratio = jnp.maximum(p / q, eps)
    r_ref[...] = jnp.log(ratio)


def _conv_kernel(xt_ref, rb_ref, z_ref, s_ref, col0, col1):
    @pl.when(pl.program_id(0) == 0)
    def _init():
        col0[...] = jnp.zeros_like(col0)
        col1[...] = jnp.zeros_like(col1)

    rb = rb_ref[...]
    smax = None
    for r in range(FF):
        col = (col0, col1)[r % 2]
        xt = xt_ref[0, r]
        for k2 in range(KB):
            col[k2 * ROWPAD:k2 * ROWPAD + AA, :] = xt[:, k2:k2 + PP + KB]
        y = jax.lax.dot_general(
            col[...], rb,
            (((0,), (0,)), ((), ())),
            preferred_element_type=jnp.float32)
        z = y[0:PP, 0:UU] + y[KB:KB + PP, 128:128 + UU]
        z_ref[0, r] = z
        m = jnp.max(z, axis=0)
        smax = m if smax is None else jnp.maximum(smax, m)
    s_ref[0, 0] = smax


def kernel(X, P_logit, Q):
    T, N, F, L, A = X.shape
    B2 = T * N

    R = pl.pallas_call(
        _prep_kernel,
        out_shape=jax.ShapeDtypeStruct((KTAPS, A, UU), jnp.float32),
    )(P_logit, Q.reshape(1, A, 1))

    Rh = R.reshape(NG, KB, A, UU)
    Rh = jnp.pad(Rh, ((0, 0), (0, 0), (0, ROWPAD - A), (0, 0)))
    Rh = Rh.reshape(NG, KB * ROWPAD, UU)
    Rh = jnp.pad(Rh, ((0, 0), (0, 256 - KB * ROWPAD), (0, 128 - UU)))
    Rbig = jnp.concatenate([Rh[0], Rh[1]], axis=1).astype(jnp.bfloat16)

    Xt = jnp.transpose(X.reshape(B2, F, L, A), (0, 1, 3, 2)).astype(jnp.bfloat16)

    Z4, S3 = pl.pallas_call(
        _conv_kernel,
        grid=(B2,),
        in_specs=[
            pl.BlockSpec((1, F, A, L), lambda i: (i, 0, 0, 0)),
            pl.BlockSpec((256, 256), lambda i: (0, 0)),
        ],
        out_specs=[
            pl.BlockSpec((1, F, PP, UU), lambda i: (i, 0, 0, 0)),
            pl.BlockSpec((1, 1, UU), lambda i: (i, 0, 0)),
        ],
        out_shape=[
            jax.ShapeDtypeStruct((B2, F, PP, UU), jnp.float32),
            jax.ShapeDtypeStruct((B2, 1, UU), jnp.float32),
        ],
        scratch_shapes=[pltpu.VMEM((256, PP + KB), jnp.bfloat16),
                        pltpu.VMEM((256, PP + KB), jnp.bfloat16)],
        compiler_params=pltpu.CompilerParams(
            dimension_semantics=("parallel",)),
    )(Xt, Rbig)

    S = S3.reshape(T, N, UU)
    Z = Z4.reshape(T, N, F, PP, UU)
    return (S, R, Z)
